# Initial kernel scaffold; baseline (speedup 1.0000x reference)
import jax
import jax.numpy as jnp
from jax.experimental import pallas as pl


def kernel(x, xa, weights, alpha, n, C):
    raise NotImplementedError("write your pallas kernel here")


# speedup vs baseline: 21.6256x; 21.6256x over previous
import jax
import jax.numpy as jnp
import numpy as np
from jax.experimental import pallas as pl

B, N, E, C, NG, L = 2, 8192, 256, 4, 8, 2
CE = E // C
M = N // NG
G = 128
NB = M // G
H = NG
HR = H * NG
MOM = 0.9

_SEG = np.arange(E) // CE
_SEL1 = (np.arange(2 * C)[:, None] == _SEG[None, :]).astype(np.float32)
_SEL2 = ((np.arange(2 * C)[:, None] - C) == _SEG[None, :]).astype(np.float32)
_OH = (_SEG[:, None] == np.arange(C)[None, :]).astype(np.float32)


def _make_bd(weights):
    w1 = weights[:, :E]
    w2 = weights[:, E:]
    oh = jnp.asarray(_OH)
    bd1 = w1[:, :, None] * oh[None]
    bd2 = w2[:, :, None] * oh[None]
    return jnp.concatenate([bd1, bd2], axis=-1)


def _layer(xi, xa, bd, sel1, sel2, alpha, m):
    for j in range(NG):
        x3 = xi.reshape(m, NG, E)
        xj = x3[:, j, :]
        r_all = jnp.dot(xi, bd, preferred_element_type=jnp.float32)
        pb = jnp.dot(r_all, sel1, preferred_element_type=jnp.float32)
        r_j = jnp.dot(xj, bd, preferred_element_type=jnp.float32)
        qb = jnp.dot(r_j, sel2, preferred_element_type=jnp.float32)
        wb = pb.reshape(m, NG, E) + qb[:, None, :]
        t = alpha * x3 + (1.0 - alpha) * wb * xj[:, None, :]
        fv = jnp.tanh(t).reshape(m * NG, E)
        xa = xa * MOM + (1.0 - MOM) * fv
        xi = xi * MOM + (1.0 - MOM) * xa
    return xi, xa


def _body(x_ref, xt_ref, xh_ref, xa_ref, xat_ref, xah_ref,
          bd_ref, sel1_ref, sel2_ref, alpha_ref, y_ref, ya_ref):
    alpha = alpha_ref[0, 0]
    sel1 = sel1_ref[...]
    sel2 = sel2_ref[...]

    xi = jnp.concatenate([xt_ref[0, 0], x_ref[0], xh_ref[0, 0]], axis=0)
    xa = jnp.concatenate([xat_ref[0, 0], xa_ref[0], xah_ref[0, 0]], axis=0)

    xi, xa = _layer(xi, xa, bd_ref[0], sel1, sel2, alpha, G + 2 * H)

    def skew_fwd(a):
        a3 = a.reshape(G + 2 * H, NG, E)
        cols = [a3[o:o + G + H, o, :][:, None, :] for o in range(NG)]
        return jnp.concatenate(cols, axis=1).reshape((G + H) * NG, E)

    xi, xa = skew_fwd(xi), skew_fwd(xa)
    xi, xa = _layer(xi, xa, bd_ref[1], sel1, sel2, alpha, G + H)

    def skew_inv(a):
        a3 = a.reshape(G + H, NG, E)
        cols = [a3[H - o:H - o + G, o, :][:, None, :] for o in range(NG)]
        return jnp.concatenate(cols, axis=1).reshape(G * NG, E)

    y_ref[0] = skew_inv(xi)
    ya_ref[0] = skew_inv(xa)


def kernel(x, xa, weights, alpha, n, C_):
    del n, C_
    bd = _make_bd(weights.astype(jnp.float32))
    alpha_arr = jnp.asarray(alpha, jnp.float32).reshape(1, 1)
    sel1 = jnp.asarray(_SEL1)
    sel2 = jnp.asarray(_SEL2)

    x4 = x.reshape(B, NB, G * NG, E)
    xa4 = xa.reshape(B, NB, G * NG, E)
    xh, xt = x4[:, :, :HR, :], x4[:, :, -HR:, :]
    xah, xat = xa4[:, :, :HR, :], xa4[:, :, -HR:, :]

    blk = pl.BlockSpec((1, G * NG, E), lambda b, g: (b, g, 0))
    prev_tail = pl.BlockSpec((1, 1, HR, E), lambda b, g: (b, (g - 1) % NB, 0, 0))
    next_head = pl.BlockSpec((1, 1, HR, E), lambda b, g: (b, (g + 1) % NB, 0, 0))
    rep2 = pl.BlockSpec((2 * C, E), lambda b, g: (0, 0))

    y, ya = pl.pallas_call(
        _body,
        grid=(B, NB),
        in_specs=[
            blk, prev_tail, next_head,
            blk, prev_tail, next_head,
            pl.BlockSpec((L, E, 2 * C), lambda b, g: (0, 0, 0)),
            rep2, rep2,
            pl.BlockSpec((1, 1), lambda b, g: (0, 0)),
        ],
        out_specs=[blk, blk],
        out_shape=[
            jax.ShapeDtypeStruct((B, N, E), jnp.float32),
            jax.ShapeDtypeStruct((B, N, E), jnp.float32),
        ],
    )(x, xt, xh, xa, xat, xah, bd, sel1, sel2, alpha_arr)
    return y, ya
